# baseline (speedup 1.0000x reference)
import functools

import jax
import jax.numpy as jnp
from jax import lax
from jax.experimental import pallas as pl
from jax.experimental.pallas import tpu as pltpu
from jax.experimental.pallas import tpu_sc as plsc


def _make_gather(B, V2, D2):
    D = D2 // 2
    info = plsc.get_sparse_core_info()
    NC, NS = info.num_cores, info.num_subcores
    NW = NC * NS
    assert B % (8 * NW) == 0
    b_per_w = B // NW
    C = 128
    nchunk = b_per_w // C
    mesh = plsc.VectorSubcoreMesh(core_axis_name="c", subcore_axis_name="s")

    @functools.partial(
        pl.kernel,
        mesh=mesh,
        out_type=jax.ShapeDtypeStruct((B, D), jnp.float32),
        scratch_types=[
            pltpu.VMEM((b_per_w,), jnp.int32),
            pltpu.VMEM((b_per_w,), jnp.int32),
            pltpu.VMEM((C, D2), jnp.float32),
            pltpu.VMEM((C, D2), jnp.float32),
            pltpu.VMEM((b_per_w, D), jnp.float32),
            pltpu.SemaphoreType.DMA,
        ],
    )
    def gather_kernel(
        xs_hbm, tablep_hbm, out_hbm, idx_v, pair_v, pairs0, pairs1, rows_v, sem
    ):
        wid = lax.axis_index("s") * NC + lax.axis_index("c")
        base = wid * b_per_w
        bufs = (pairs0, pairs1)
        pltpu.sync_copy(xs_hbm.at[pl.ds(base, b_per_w)], idx_v)

        def to_pairs(c, carry):
            off = c * 16
            idx_v16 = idx_v[pl.ds(off, 16)]
            pair_v[pl.ds(off, 16)] = lax.shift_right_logical(idx_v16, 1)
            return carry

        lax.fori_loop(0, b_per_w // 16, to_pairs, 0)

        pltpu.async_copy(
            tablep_hbm.at[pair_v.at[pl.ds(0, C)]], pairs0, sem
        )
        for c in range(nchunk):
            buf = bufs[c % 2]
            pltpu.make_async_copy(
                tablep_hbm.at[pl.ds(0, C)], buf, sem
            ).wait()
            if c + 1 < nchunk:
                pltpu.async_copy(
                    tablep_hbm.at[pair_v.at[pl.ds((c + 1) * C, C)]],
                    bufs[(c + 1) % 2],
                    sem,
                )

            def select(cc, carry):
                off = cc * 16
                vec = idx_v[pl.ds(c * C + off, 16)]
                for j in range(16):
                    h = (vec[j] & 1) * D
                    for k in range(D // 16):
                        rows_v[c * C + off + j, pl.ds(k * 16, 16)] = buf[
                            off + j, pl.ds(h + k * 16, 16)
                        ]
                return carry

            lax.fori_loop(0, C // 16, select, 0)

        pltpu.sync_copy(rows_v, out_hbm.at[pl.ds(base, b_per_w)])

    return gather_kernel


@jax.jit
def kernel(xs, table):
    B = xs.shape[0]
    V, D = table.shape
    tablep = jnp.reshape(table, (V // 2, 2 * D))
    return _make_gather(B, V // 2, 2 * D)(xs.astype(jnp.int32), tablep)


# speedup vs baseline: 1.7332x; 1.7332x over previous
import functools

import jax
import jax.numpy as jnp
from jax import lax
from jax.experimental import pallas as pl
from jax.experimental.pallas import tpu as pltpu
from jax.experimental.pallas import tpu_sc as plsc


def _make_gather(B, V, D):
    info = plsc.get_sparse_core_info()
    NC, NS = info.num_cores, info.num_subcores
    NW = NC * NS
    assert B % (8 * NW) == 0
    b_per_w = B // NW
    mesh = plsc.VectorSubcoreMesh(core_axis_name="c", subcore_axis_name="s")

    @functools.partial(
        pl.kernel,
        mesh=mesh,
        out_type=jax.ShapeDtypeStruct((B, D), jnp.float32),
        scratch_types=[
            pltpu.VMEM((b_per_w,), jnp.int32),
            pltpu.VMEM((b_per_w, D), jnp.float32),
            pltpu.SemaphoreType.DMA,
        ],
    )
    def gather_kernel(xs_hbm, table_hbm, out_hbm, idx_v, rows_v, sem):
        wid = lax.axis_index("s") * NC + lax.axis_index("c")
        base = wid * b_per_w
        pltpu.sync_copy(xs_hbm.at[pl.ds(base, b_per_w)], idx_v)

        def fire(c, carry):
            off = c * 16
            vec = idx_v[pl.ds(off, 16)]
            for j in range(16):
                pltpu.async_copy(
                    table_hbm.at[pl.ds(vec[j], 1), :],
                    rows_v.at[pl.ds(off + j, 1), :],
                    sem,
                )
            return carry

        lax.fori_loop(0, b_per_w // 16, fire, 0)
        pltpu.make_async_copy(
            table_hbm.at[pl.ds(0, b_per_w), :], rows_v, sem
        ).wait()
        pltpu.sync_copy(rows_v, out_hbm.at[pl.ds(base, b_per_w)])

    return gather_kernel


@jax.jit
def kernel(xs, table):
    B = xs.shape[0]
    V, D = table.shape
    return _make_gather(B, V, D)(xs.astype(jnp.int32), table)
